# Initial kernel scaffold; baseline (speedup 1.0000x reference)
import jax
import jax.numpy as jnp
from jax.experimental import pallas as pl


def kernel(sparse_0, sparse_1, sparse_2, sparse_3, sparse_4, sparse_5, sparse_6, sparse_7, sparse_8, sparse_9, sparse_10, sparse_11, sparse_12, sparse_13, sparse_14, sparse_15, sparse_16, sparse_17, sparse_18, sparse_19, sparse_20, sparse_21, sparse_22, sparse_23, sparse_24, sparse_25, dense_0, dense_1, dense_2, dense_3, dense_4, dense_5, dense_6, dense_7, dense_8, dense_9, dense_10, dense_11, dense_12, W_0, W_1, W_2, W_3, W_4, W_5, W_6, W_7, W_8, W_9, W_10, W_11, W_12, W_13, W_14, W_15, W_16, W_17, W_18, W_19, W_20, W_21, W_22, W_23, W_24, W_25, Wd):
    raise NotImplementedError("write your pallas kernel here")


# speedup vs baseline: 2.5136x; 2.5136x over previous
import functools

import jax
import jax.numpy as jnp
from jax import lax
from jax.experimental import pallas as pl
from jax.experimental.pallas import tpu as pltpu
from jax.experimental.pallas import tpu_sc as plsc

NS = 26
ND = 13
B = 16384
V = 100000
NC = 2
NSUB = 16
NW = NC * NSUB
BPW = B // NW
ROWS = BPW // 128
CHUNKS = BPW // 16

_mesh = plsc.VectorSubcoreMesh(core_axis_name="c", subcore_axis_name="s")


@functools.partial(
    pl.kernel,
    mesh=_mesh,
    out_type=jax.ShapeDtypeStruct((B // 128, 128), jnp.float32),
    scratch_types=[
        pltpu.VMEM((NS, ROWS, 128), jnp.int32),
        pltpu.VMEM((NS, ROWS, 128), jnp.float32),
        pltpu.VMEM((ND, ROWS, 128), jnp.float32),
        pltpu.VMEM((ND, 16), jnp.float32),
        pltpu.VMEM((ROWS, 128), jnp.float32),
        pltpu.SemaphoreType.DMA,
        pltpu.SemaphoreType.DMA,
    ],
)
def _sc_kernel(idx_hbm, dense_hbm, wd_hbm, *rest):
    tables = rest[:NS]
    out_hbm = rest[NS]
    idx_v, gath_v, dense_v, wd_v, acc_v, sem_in, sem_g = rest[NS + 1:]

    wid = lax.axis_index("s") * NC + lax.axis_index("c")
    base = wid * ROWS

    copies = []
    copies.append(pltpu.async_copy(wd_hbm, wd_v, sem_in))
    for i in range(NS):
        copies.append(
            pltpu.async_copy(idx_hbm.at[i, pl.ds(base, ROWS)], idx_v.at[i], sem_in)
        )
    for d in range(ND):
        copies.append(
            pltpu.async_copy(dense_hbm.at[d, pl.ds(base, ROWS)], dense_v.at[d], sem_in)
        )
    for c in copies:
        c.wait()

    gathers = []
    for i in range(NS):
        for r in range(ROWS):
            gathers.append(
                pltpu.async_copy(tables[i].at[idx_v.at[i, r]], gath_v.at[i, r], sem_g)
            )
    for c in gathers:
        c.wait()

    for c in range(CHUNKS):
        r, o = c // 8, (c % 8) * 16
        acc = gath_v[0, r, pl.ds(o, 16)]
        for i in range(1, NS):
            acc = acc + gath_v[i, r, pl.ds(o, 16)]
        for d in range(ND):
            acc = acc + dense_v[d, r, pl.ds(o, 16)] * wd_v[d, :]
        acc_v[r, pl.ds(o, 16)] = acc

    pltpu.sync_copy(acc_v, out_hbm.at[pl.ds(base, ROWS)])


def kernel(sparse_0, sparse_1, sparse_2, sparse_3, sparse_4, sparse_5, sparse_6, sparse_7, sparse_8, sparse_9, sparse_10, sparse_11, sparse_12, sparse_13, sparse_14, sparse_15, sparse_16, sparse_17, sparse_18, sparse_19, sparse_20, sparse_21, sparse_22, sparse_23, sparse_24, sparse_25, dense_0, dense_1, dense_2, dense_3, dense_4, dense_5, dense_6, dense_7, dense_8, dense_9, dense_10, dense_11, dense_12, W_0, W_1, W_2, W_3, W_4, W_5, W_6, W_7, W_8, W_9, W_10, W_11, W_12, W_13, W_14, W_15, W_16, W_17, W_18, W_19, W_20, W_21, W_22, W_23, W_24, W_25, Wd):
    sparse = [sparse_0, sparse_1, sparse_2, sparse_3, sparse_4, sparse_5,
              sparse_6, sparse_7, sparse_8, sparse_9, sparse_10, sparse_11,
              sparse_12, sparse_13, sparse_14, sparse_15, sparse_16,
              sparse_17, sparse_18, sparse_19, sparse_20, sparse_21,
              sparse_22, sparse_23, sparse_24, sparse_25]
    dense = [dense_0, dense_1, dense_2, dense_3, dense_4, dense_5, dense_6,
             dense_7, dense_8, dense_9, dense_10, dense_11, dense_12]
    tables = [W_0, W_1, W_2, W_3, W_4, W_5, W_6, W_7, W_8, W_9, W_10, W_11,
              W_12, W_13, W_14, W_15, W_16, W_17, W_18, W_19, W_20, W_21,
              W_22, W_23, W_24, W_25]

    idx3 = jnp.stack([s[:, 0] for s in sparse]).reshape(NS, B // 128, 128)
    dense3 = jnp.stack([d[:, 0] for d in dense]).reshape(ND, B // 128, 128)
    wd16 = jnp.broadcast_to(Wd, (ND, 16))
    flat_tables = [w[:, 0] for w in tables]

    out = _sc_kernel(idx3, dense3, wd16, *flat_tables)
    return out.reshape(B, 1)
